# Initial kernel scaffold; baseline (speedup 1.0000x reference)
import jax
import jax.numpy as jnp
from jax.experimental import pallas as pl


def kernel(x, edge_index, W1, b1, W2, b2, W_out, b_out):
    raise NotImplementedError("write your pallas kernel here")


# speedup vs baseline: 31.4227x; 31.4227x over previous
import functools

import jax
import jax.numpy as jnp
from jax import lax
from jax.experimental import pallas as pl
from jax.experimental.pallas import tpu as pltpu
from jax.experimental.pallas import tpu_sc as plsc

N_NODES = 10000
D_FEAT = 128
N_PAD = 10240
PAD_ROW = N_NODES
N_EDGES_SELF = 330000
BLK = 128
K_BLK = 81
E_PER_TILE = K_BLK * BLK
E_PAD = E_PER_TILE * 32
ROWS_PER_TILE = N_PAD // 16

_mesh = lambda: plsc.VectorSubcoreMesh(core_axis_name="c", subcore_axis_name="s")
_sc_params = lambda: pltpu.CompilerParams(use_tc_tiling_on_sc=False)


def _deg_kernel():
    @functools.partial(
        pl.kernel,
        mesh=_mesh(),
        out_type=jax.ShapeDtypeStruct((2 * N_PAD,), jnp.float32),
        compiler_params=_sc_params(),
        scratch_types=[
            pltpu.VMEM((K_BLK, BLK), jnp.int32),
            pltpu.VMEM((BLK,), jnp.float32),
            pltpu.VMEM((ROWS_PER_TILE,), jnp.float32),
            pltpu.VMEM_SHARED((N_PAD,), jnp.float32),
        ],
    )
    def degk(dst_hbm, out_hbm, dst_v, ones_v, zero_v, acc_sh):
        c = lax.axis_index("c")
        s = lax.axis_index("s")
        pltpu.sync_copy(dst_hbm.at[c * 16 + s], dst_v)
        ones16 = jnp.ones((16,), jnp.float32)
        zero16 = jnp.zeros((16,), jnp.float32)
        for i in range(BLK // 16):
            ones_v[pl.ds(i * 16, 16)] = ones16

        def zb(i, carry):
            zero_v[pl.ds(i * 16, 16)] = zero16
            return carry

        lax.fori_loop(0, ROWS_PER_TILE // 16, zb, 0)
        pltpu.sync_copy(zero_v, acc_sh.at[pl.ds(s * ROWS_PER_TILE, ROWS_PER_TILE)])
        plsc.subcore_barrier()

        def body(j, carry):
            pltpu.sync_copy(ones_v, acc_sh.at[dst_v.at[j]], add=True)
            return carry

        lax.fori_loop(0, K_BLK, body, 0)
        plsc.subcore_barrier()
        pltpu.sync_copy(
            acc_sh.at[pl.ds(s * ROWS_PER_TILE, ROWS_PER_TILE)],
            out_hbm.at[pl.ds(c * N_PAD + s * ROWS_PER_TILE, ROWS_PER_TILE)],
        )

    return degk


def _agg_kernel(d):
    @functools.partial(
        pl.kernel,
        mesh=_mesh(),
        out_type=jax.ShapeDtypeStruct((2, N_PAD, d), jnp.float32),
        compiler_params=_sc_params(),
        scratch_types=[
            pltpu.VMEM((K_BLK, BLK), jnp.int32),
            pltpu.VMEM((K_BLK, BLK), jnp.int32),
            pltpu.VMEM((BLK, d), jnp.float32),
            pltpu.VMEM((ROWS_PER_TILE, d), jnp.float32),
            pltpu.VMEM_SHARED((N_PAD, d), jnp.float32),
            pltpu.SemaphoreType.DMA,
        ],
    )
    def aggk(p_hbm, src_hbm, dst_hbm, out_hbm, src_v, dst_v, rows_v, zero_v, acc_sh, sem):
        c = lax.axis_index("c")
        s = lax.axis_index("s")
        wid = c * 16 + s
        pltpu.sync_copy(src_hbm.at[wid], src_v)
        pltpu.sync_copy(dst_hbm.at[wid], dst_v)
        zero16 = jnp.zeros((16,), jnp.float32)

        def zb(r, carry):
            for cc in range(d // 16):
                zero_v[r, pl.ds(cc * 16, 16)] = zero16
            return carry

        lax.fori_loop(0, ROWS_PER_TILE, zb, 0)
        pltpu.sync_copy(zero_v, acc_sh.at[pl.ds(s * ROWS_PER_TILE, ROWS_PER_TILE)])
        plsc.subcore_barrier()

        def body(j, carry):
            pltpu.async_copy(p_hbm.at[src_v.at[j]], rows_v, sem).wait()
            pltpu.sync_copy(rows_v, acc_sh.at[dst_v.at[j]], add=True)
            return carry

        lax.fori_loop(0, K_BLK, body, 0)
        plsc.subcore_barrier()
        pltpu.sync_copy(
            acc_sh.at[pl.ds(s * ROWS_PER_TILE, ROWS_PER_TILE)],
            out_hbm.at[c, pl.ds(s * ROWS_PER_TILE, ROWS_PER_TILE)],
        )

    return aggk


def _dinv_of(deg_t):
    deg = deg_t[:, 0:1] + deg_t[:, 1:2]
    return jnp.where(deg > 0.0, lax.rsqrt(deg), 0.0)


def _tc_stage_b(deg_ref, x_ref, w1_ref, p1_ref):
    dinv = _dinv_of(deg_ref[...])
    u = jnp.dot(x_ref[...], w1_ref[...], preferred_element_type=jnp.float32)
    p1_ref[...] = u * dinv


def _tc_stage_d(parts_ref, deg_ref, b1_ref, w2_ref, p2_ref):
    dinv = _dinv_of(deg_ref[...])
    agg = parts_ref[0] + parts_ref[1]
    h1 = jnp.maximum(agg * dinv + b1_ref[...], 0.0)
    p2_ref[...] = jnp.dot(h1 * dinv, w2_ref[...], preferred_element_type=jnp.float32)


def _tc_stage_f(parts_ref, deg_ref, b2_ref, wout_ref, bout_ref, out_ref):
    dinv = _dinv_of(deg_ref[...])
    h2 = jnp.maximum((parts_ref[0] + parts_ref[1]) * dinv + b2_ref[...], 0.0)
    out_ref[...] = (
        jnp.dot(h2, wout_ref[...], preferred_element_type=jnp.float32) + bout_ref[...]
    )


def kernel(x, edge_index, W1, b1, W2, b2, W_out, b_out):
    n = x.shape[0]
    loop = jnp.arange(n, dtype=edge_index.dtype)
    pad = jnp.full((E_PAD - N_EDGES_SELF,), PAD_ROW, dtype=edge_index.dtype)
    src = jnp.concatenate([edge_index[0], loop, pad]).reshape(32, K_BLK, BLK)
    dst = jnp.concatenate([edge_index[1], loop, pad]).reshape(32, K_BLK, BLK)
    x_pad = jnp.zeros((N_PAD, D_FEAT), x.dtype).at[:n].set(x)

    deg_parts = _deg_kernel()(dst).reshape(2, N_PAD)
    deg_t = deg_parts.T

    p1 = pl.pallas_call(
        _tc_stage_b,
        out_shape=jax.ShapeDtypeStruct((N_PAD, 16), jnp.float32),
    )(deg_t, x_pad, W1)

    agg1 = _agg_kernel(16)(p1, src, dst)

    p2 = pl.pallas_call(
        _tc_stage_d,
        out_shape=jax.ShapeDtypeStruct((N_PAD, 32), jnp.float32),
    )(agg1, deg_t, b1, W2)

    agg2 = _agg_kernel(32)(p2, src, dst)

    out2d = pl.pallas_call(
        _tc_stage_f,
        out_shape=jax.ShapeDtypeStruct((N_PAD, D_FEAT), jnp.float32),
    )(agg2, deg_t, b2, W_out, b_out)

    return out2d[:n].reshape(n // 10, 10, D_FEAT)
